# baseline (speedup 1.0000x reference)
import functools

import jax
import jax.numpy as jnp
from jax import lax
from jax.experimental import pallas as pl
from jax.experimental.pallas import tpu as pltpu
from jax.experimental.pallas import tpu_sc as plsc

_X_MIN = -5.0
_X_MAX = 5.0
_NBINS = 256

_NC = 2
_NS = 16
_NW = _NC * _NS
_LANES = 16

_ROWS = 4096
_COLS = 8192
_ROWS_W = _ROWS // _NW
_BR = 8
_BC = 2048
_NCHUNKS = (_ROWS_W // _BR) * (_COLS // _BC)
_CPW = _COLS // _BC
_NBUF = 4
_UNROLL = 16


def _hist_body(x_hbm, part_hbm, xout_hbm,
               buf0, buf1, buf2, buf3, hist, lhist,
               si0, si1, si2, si3, so0, so1, so2, so3):
    cid = lax.axis_index("c")
    sid = lax.axis_index("s")
    wid = sid * _NC + cid
    row0 = wid * _ROWS_W

    bufs = (buf0, buf1, buf2, buf3)
    sins = (si0, si1, si2, si3)
    souts = (so0, so1, so2, so3)

    zeros16 = jnp.zeros((_LANES,), jnp.int32)

    def zero_seg(t, _):
        hist[pl.ds(t * _LANES, _LANES)] = zeros16
        return 0

    lax.fori_loop(0, _NS * _NBINS // _LANES, zero_seg, 0)

    lane_base = (lax.iota(jnp.int32, _LANES) * _NBINS).astype(jnp.float32)
    ones = jnp.ones((_LANES,), jnp.int32)
    scale = jnp.float32(_NBINS / (_X_MAX - _X_MIN))
    shiftv = lane_base + jnp.float32(-_X_MIN * _NBINS / (_X_MAX - _X_MIN))
    lov = lane_base
    hiv = lane_base + jnp.float32(_NBINS - 1)

    def blk_slice(ref, ci):
        rb = ci // _CPW
        cb = ci % _CPW
        return ref.at[pl.ds(row0 + rb * _BR, _BR), pl.ds(cb * _BC, _BC)]

    def copy_in(ci, s):
        return pltpu.make_async_copy(blk_slice(x_hbm, ci), bufs[s], sins[s])

    def copy_out(ci, s):
        return pltpu.make_async_copy(
            bufs[s], blk_slice(xout_hbm, ci), souts[s])

    def compute(bref):
        def row_body(r, _):
            def vec_body(j, _):
                b = j * (_LANES * _UNROLL)
                vs = [bref[r, pl.ds(b + u * _LANES, _LANES)]
                      for u in range(_UNROLL)]
                idxs = []
                for v in vs:
                    t = v * scale + shiftv
                    t = jnp.minimum(jnp.maximum(t, lov), hiv)
                    idxs.append(t.astype(jnp.int32))
                for ix in idxs:
                    plsc.addupdate_scatter(hist, [ix], ones)
                return 0

            lax.fori_loop(0, _BC // (_LANES * _UNROLL), vec_body, 0)
            return 0

        lax.fori_loop(0, _BR, row_body, 0)

    copy_in(0, 0).start()
    copy_in(1, 1).start()

    def quad_body(q, _):
        k0 = _NBUF * q
        for s in range(_NBUF):
            k = k0 + s
            t = (s + 2) % _NBUF
            copy_in(k, s).wait()

            @pl.when(k >= 2)
            def _():
                copy_out(k - 2, t).wait()

            @pl.when(k + 2 < _NCHUNKS)
            def _():
                copy_in(k + 2, t).start()

            compute(bufs[s])
            copy_out(k, s).start()
        return 0

    lax.fori_loop(0, _NCHUNKS // _NBUF, quad_body, 0)
    copy_out(_NCHUNKS - 2, (_NCHUNKS - 2) % _NBUF).wait()
    copy_out(_NCHUNKS - 1, (_NCHUNKS - 1) % _NBUF).wait()

    def red_seg(t, _):
        acc = hist[pl.ds(t * _LANES, _LANES)]
        for r in range(1, _NS):
            acc = acc + hist[pl.ds(r * _NBINS + t * _LANES, _LANES)]
        lhist[pl.ds(t * _LANES, _LANES)] = acc
        return 0

    lax.fori_loop(0, _NBINS // _LANES, red_seg, 0)
    pltpu.sync_copy(lhist, part_hbm.at[wid])


@functools.partial(jax.jit)
def _hist_and_copy(x):
    mesh = plsc.VectorSubcoreMesh(
        core_axis_name="c", subcore_axis_name="s",
        num_cores=_NC, num_subcores=_NS)
    partials, xout = pl.kernel(
        _hist_body,
        out_type=(
            jax.ShapeDtypeStruct((_NW, _NBINS), jnp.int32),
            jax.ShapeDtypeStruct((_ROWS, _COLS), jnp.float32),
        ),
        mesh=mesh,
        compiler_params=pltpu.CompilerParams(
            needs_layout_passes=False, use_tc_tiling_on_sc=True),
        scratch_types=[
            pltpu.VMEM((_BR, _BC), jnp.float32),
            pltpu.VMEM((_BR, _BC), jnp.float32),
            pltpu.VMEM((_BR, _BC), jnp.float32),
            pltpu.VMEM((_BR, _BC), jnp.float32),
            pltpu.VMEM((_NS * _NBINS,), jnp.int32),
            pltpu.VMEM((_NBINS,), jnp.int32),
            pltpu.SemaphoreType.DMA,
            pltpu.SemaphoreType.DMA,
            pltpu.SemaphoreType.DMA,
            pltpu.SemaphoreType.DMA,
            pltpu.SemaphoreType.DMA,
            pltpu.SemaphoreType.DMA,
            pltpu.SemaphoreType.DMA,
            pltpu.SemaphoreType.DMA,
        ],
    )(x)
    return jnp.sum(partials, axis=0), xout


def kernel(x):
    hist, xout = _hist_and_copy(x)
    return (xout, hist.astype(jnp.int64))


# speedup vs baseline: 1.2401x; 1.2401x over previous
import functools

import jax
import jax.numpy as jnp
from jax import lax
from jax.experimental import pallas as pl
from jax.experimental.pallas import tpu as pltpu
from jax.experimental.pallas import tpu_sc as plsc

_X_MIN = -5.0
_X_MAX = 5.0
_NBINS = 256

_NC = 2
_NS = 16
_NW = _NC * _NS
_LANES = 16

_ROWS = 4096
_COLS = 8192
_ROWS_W = _ROWS // _NW
_BR = 8
_BC = 2048
_NCHUNKS = (_ROWS_W // _BR) * (_COLS // _BC)
_CPW = _COLS // _BC
_NBUF = 4
_UNROLL = 32


def _hist_body(x_hbm, part_hbm, xout_hbm,
               buf0, buf1, buf2, buf3, hist, lhist,
               si0, si1, si2, si3, so0, so1, so2, so3):
    cid = lax.axis_index("c")
    sid = lax.axis_index("s")
    wid = sid * _NC + cid
    row0 = wid * _ROWS_W

    bufs = (buf0, buf1, buf2, buf3)
    sins = (si0, si1, si2, si3)
    souts = (so0, so1, so2, so3)

    zeros16 = jnp.zeros((_LANES,), jnp.int32)

    def zero_seg(t, _):
        hist[pl.ds(t * _LANES, _LANES)] = zeros16
        return 0

    lax.fori_loop(0, _NS * _NBINS // _LANES, zero_seg, 0)

    lane_base = (lax.iota(jnp.int32, _LANES) * _NBINS).astype(jnp.float32)
    ones = jnp.ones((_LANES,), jnp.int32)
    scale = jnp.float32(_NBINS / (_X_MAX - _X_MIN))
    shiftv = lane_base + jnp.float32(-_X_MIN * _NBINS / (_X_MAX - _X_MIN))
    lov = lane_base
    hiv = lane_base + jnp.float32(_NBINS - 1)

    def blk_slice(ref, ci):
        rb = ci // _CPW
        cb = ci % _CPW
        return ref.at[pl.ds(row0 + rb * _BR, _BR), pl.ds(cb * _BC, _BC)]

    def copy_in(ci, s):
        return pltpu.make_async_copy(blk_slice(x_hbm, ci), bufs[s], sins[s])

    def copy_out(ci, s):
        return pltpu.make_async_copy(
            bufs[s], blk_slice(xout_hbm, ci), souts[s])

    def compute(bref):
        def row_body(r, _):
            def vec_body(j, _):
                b = j * (_LANES * _UNROLL)
                vs = [bref[r, pl.ds(b + u * _LANES, _LANES)]
                      for u in range(_UNROLL)]
                idxs = []
                for v in vs:
                    t = v * scale + shiftv
                    t = jnp.minimum(jnp.maximum(t, lov), hiv)
                    idxs.append(t.astype(jnp.int32))
                for ix in idxs:
                    plsc.addupdate_scatter(hist, [ix], ones)
                return 0

            lax.fori_loop(0, _BC // (_LANES * _UNROLL), vec_body, 0)
            return 0

        lax.fori_loop(0, _BR, row_body, 0)

    copy_in(0, 0).start()
    copy_in(1, 1).start()

    def quad_body(q, _):
        k0 = _NBUF * q
        for s in range(_NBUF):
            k = k0 + s
            t = (s + 2) % _NBUF
            copy_in(k, s).wait()

            @pl.when(k >= 2)
            def _():
                copy_out(k - 2, t).wait()

            @pl.when(k + 2 < _NCHUNKS)
            def _():
                copy_in(k + 2, t).start()

            compute(bufs[s])
            copy_out(k, s).start()
        return 0

    lax.fori_loop(0, _NCHUNKS // _NBUF, quad_body, 0)
    copy_out(_NCHUNKS - 2, (_NCHUNKS - 2) % _NBUF).wait()
    copy_out(_NCHUNKS - 1, (_NCHUNKS - 1) % _NBUF).wait()

    def red_seg(t, _):
        acc = hist[pl.ds(t * _LANES, _LANES)]
        for r in range(1, _NS):
            acc = acc + hist[pl.ds(r * _NBINS + t * _LANES, _LANES)]
        lhist[pl.ds(t * _LANES, _LANES)] = acc
        return 0

    lax.fori_loop(0, _NBINS // _LANES, red_seg, 0)
    pltpu.sync_copy(lhist, part_hbm.at[wid])


@functools.partial(jax.jit)
def _hist_and_copy(x):
    mesh = plsc.VectorSubcoreMesh(
        core_axis_name="c", subcore_axis_name="s",
        num_cores=_NC, num_subcores=_NS)
    partials, xout = pl.kernel(
        _hist_body,
        out_type=(
            jax.ShapeDtypeStruct((_NW, _NBINS), jnp.int32),
            jax.ShapeDtypeStruct((_ROWS, _COLS), jnp.float32),
        ),
        mesh=mesh,
        compiler_params=pltpu.CompilerParams(
            needs_layout_passes=False, use_tc_tiling_on_sc=True),
        scratch_types=[
            pltpu.VMEM((_BR, _BC), jnp.float32),
            pltpu.VMEM((_BR, _BC), jnp.float32),
            pltpu.VMEM((_BR, _BC), jnp.float32),
            pltpu.VMEM((_BR, _BC), jnp.float32),
            pltpu.VMEM((_NS * _NBINS,), jnp.int32),
            pltpu.VMEM((_NBINS,), jnp.int32),
            pltpu.SemaphoreType.DMA,
            pltpu.SemaphoreType.DMA,
            pltpu.SemaphoreType.DMA,
            pltpu.SemaphoreType.DMA,
            pltpu.SemaphoreType.DMA,
            pltpu.SemaphoreType.DMA,
            pltpu.SemaphoreType.DMA,
            pltpu.SemaphoreType.DMA,
        ],
    )(x)
    return jnp.sum(partials, axis=0), xout


def kernel(x):
    hist, xout = _hist_and_copy(x)
    return (xout, hist.astype(jnp.int64))
